# Initial kernel scaffold; baseline (speedup 1.0000x reference)
import jax
import jax.numpy as jnp
from jax.experimental import pallas as pl


def kernel(inputs, tables):
    raise NotImplementedError("write your pallas kernel here")


# speedup vs baseline: 6.7194x; 6.7194x over previous
import functools

import jax
import jax.numpy as jnp
from jax import lax
from jax.experimental import pallas as pl
from jax.experimental.pallas import tpu as pltpu
from jax.experimental.pallas import tpu_sc as plsc

_N_COLS = 26
_VOCAB = 1000
_D = 64
_GROUP = 128
_LANES = 16
_NBUF = 2


@functools.lru_cache(maxsize=None)
def _make_kernel(n_rows: int):
    info = plsc.get_sparse_core_info()
    nw = info.num_cores * info.num_subcores
    rows_per_w = n_rows // nw
    assert rows_per_w * nw == n_rows
    n_groups = rows_per_w // _GROUP
    assert n_groups * _GROUP == rows_per_w
    assert n_groups % _NBUF == 0

    mesh = plsc.VectorSubcoreMesh(core_axis_name="c", subcore_axis_name="s")
    nc = info.num_cores

    @functools.partial(
        pl.kernel,
        out_type=jax.ShapeDtypeStruct((n_rows, _D), jnp.float32),
        mesh=mesh,
        compiler_params=pltpu.CompilerParams(use_tc_tiling_on_sc=False),
        scratch_types=[
            pltpu.VMEM((rows_per_w,), jnp.int32),
            pltpu.VMEM((_GROUP, _D), jnp.float32),
            pltpu.VMEM((_GROUP, _D), jnp.float32),
            pltpu.SemaphoreType.DMA,
            pltpu.SemaphoreType.DMA,
        ],
    )
    def gather_kernel(idx_hbm, table_hbm, out_hbm, idx_v, buf0, buf1, g0, g1):
        wid = lax.axis_index("s") * nc + lax.axis_index("c")
        row_base = wid * rows_per_w

        pltpu.sync_copy(idx_hbm.at[pl.ds(row_base, rows_per_w)], idx_v)

        lanes = lax.broadcasted_iota(jnp.int32, (_LANES,), 0)

        def offset_body(j, _):
            k0 = row_base + j * _GROUP
            for i in range(_GROUP // _LANES):
                pos = lanes + (k0 + i * _LANES)
                col = lax.rem(pos, _N_COLS)
                sl = pl.ds(j * _GROUP + i * _LANES, _LANES)
                idx_v[sl] = idx_v[sl] + col * _VOCAB
            return 0

        lax.fori_loop(0, n_groups, offset_body, 0)

        bufs = (buf0, buf1)
        sems = (g0, g1)

        def start_gather(g, b):
            pltpu.async_copy(
                table_hbm.at[idx_v.at[pl.ds(g * _GROUP, _GROUP)]],
                bufs[b],
                sems[b],
            )

        def wait_gather(g, b):
            pltpu.make_async_copy(
                table_hbm.at[idx_v.at[pl.ds(g * _GROUP, _GROUP)]],
                bufs[b],
                sems[b],
            ).wait()

        for b in range(_NBUF):
            start_gather(b, b)

        def pipe_body(p, _):
            for b in range(_NBUF):
                g = p * _NBUF + b
                wait_gather(g, b)
                pltpu.sync_copy(
                    bufs[b],
                    out_hbm.at[pl.ds(row_base + g * _GROUP, _GROUP)],
                )

                @pl.when(g + _NBUF < n_groups)
                def _():
                    start_gather(g + _NBUF, b)

            return 0

        lax.fori_loop(0, n_groups // _NBUF, pipe_body, 0)

    return gather_kernel


def kernel(inputs, tables):
    b, l, c = inputs.shape
    n_rows = b * l * c
    idx2d = inputs.astype(jnp.int32).reshape(n_rows)
    flat_tables = tables.reshape(c * tables.shape[1], tables.shape[2])
    out = _make_kernel(n_rows)(idx2d, flat_tables)
    return out.reshape(b, l, c * _D)


# speedup vs baseline: 7.0494x; 1.0491x over previous
import functools

import jax
import jax.numpy as jnp
from jax import lax
from jax.experimental import pallas as pl
from jax.experimental.pallas import tpu as pltpu
from jax.experimental.pallas import tpu_sc as plsc

_N_COLS = 26
_VOCAB = 1000
_D = 64
_GROUP = 128
_LANES = 16
_RING = 10
_AHEAD = 5


@functools.lru_cache(maxsize=None)
def _make_kernel(n_rows: int):
    info = plsc.get_sparse_core_info()
    nw = info.num_cores * info.num_subcores
    rows_per_w = n_rows // nw
    assert rows_per_w * nw == n_rows
    n_groups = rows_per_w // _GROUP
    assert n_groups * _GROUP == rows_per_w
    assert n_groups % _RING == 0

    mesh = plsc.VectorSubcoreMesh(core_axis_name="c", subcore_axis_name="s")
    nc = info.num_cores

    @functools.partial(
        pl.kernel,
        out_type=jax.ShapeDtypeStruct((n_rows, _D), jnp.float32),
        mesh=mesh,
        compiler_params=pltpu.CompilerParams(use_tc_tiling_on_sc=False),
        scratch_types=[
            pltpu.VMEM((rows_per_w,), jnp.int32),
            tuple(pltpu.VMEM((_GROUP, _D), jnp.float32) for _ in range(_RING)),
            tuple(pltpu.SemaphoreType.DMA for _ in range(_RING)),
            tuple(pltpu.SemaphoreType.DMA for _ in range(_RING)),
        ],
    )
    def gather_kernel(idx_hbm, table_hbm, out_hbm, idx_v, bufs, gsems, ssems):
        wid = lax.axis_index("s") * nc + lax.axis_index("c")
        row_base = wid * rows_per_w

        pltpu.sync_copy(idx_hbm.at[pl.ds(row_base, rows_per_w)], idx_v)

        lanes = lax.broadcasted_iota(jnp.int32, (_LANES,), 0)

        def add_offsets(g):
            k0 = row_base + g * _GROUP
            for i in range(_GROUP // _LANES):
                pos = lanes + (k0 + i * _LANES)
                col = lax.rem(pos, _N_COLS)
                sl = pl.ds(g * _GROUP + i * _LANES, _LANES)
                idx_v[sl] = idx_v[sl] + col * _VOCAB

        def start_gather(g, b):
            pltpu.async_copy(
                table_hbm.at[idx_v.at[pl.ds(g * _GROUP, _GROUP)]],
                bufs[b],
                gsems[b],
            )

        def wait_gather(g, b):
            pltpu.make_async_copy(
                table_hbm.at[idx_v.at[pl.ds(g * _GROUP, _GROUP)]],
                bufs[b],
                gsems[b],
            ).wait()

        def out_slice(g):
            return out_hbm.at[pl.ds(row_base + g * _GROUP, _GROUP)]

        def start_store(g, b):
            pltpu.async_copy(bufs[b], out_slice(g), ssems[b])

        def wait_store(g, b):
            pltpu.make_async_copy(bufs[b], out_slice(g), ssems[b]).wait()

        for g in range(_AHEAD):
            add_offsets(g)
            start_gather(g, g)

        def pipe_body(p, _):
            for b in range(_RING):
                g = p * _RING + b
                wait_gather(g, b)
                start_store(g, b)
                h = g + _AHEAD

                @pl.when(h < n_groups)
                def _():
                    hb = (b + _AHEAD) % _RING

                    @pl.when(h >= _RING)
                    def _():
                        wait_store(h - _RING, hb)

                    add_offsets_dyn(h)
                    start_gather(h, hb)

            return 0

        def add_offsets_dyn(g):
            add_offsets(g)

        lax.fori_loop(0, n_groups // _RING, pipe_body, 0)

        for b in range(_RING):
            g = n_groups - _RING + b
            wait_store(g, b)

    return gather_kernel


def kernel(inputs, tables):
    b, l, c = inputs.shape
    n_rows = b * l * c
    idx_flat = inputs.astype(jnp.int32).reshape(n_rows)
    flat_tables = tables.reshape(c * tables.shape[1], tables.shape[2])
    out = _make_kernel(n_rows)(idx_flat, flat_tables)
    return out.reshape(b, l, c * _D)


# speedup vs baseline: 7.6224x; 1.0813x over previous
import functools

import jax
import jax.numpy as jnp
from jax import lax
from jax.experimental import pallas as pl
from jax.experimental.pallas import tpu as pltpu
from jax.experimental.pallas import tpu_sc as plsc

_N_COLS = 26
_VOCAB = 1000
_D = 64
_GROUP = 64
_LANES = 16
_RING = 5
_AHEAD = 3


@functools.lru_cache(maxsize=None)
def _make_kernel(n_rows: int):
    info = plsc.get_sparse_core_info()
    nw = info.num_cores * info.num_subcores
    rows_per_w = n_rows // nw
    assert rows_per_w * nw == n_rows
    n_groups = rows_per_w // _GROUP
    assert n_groups * _GROUP == rows_per_w
    assert n_groups % _RING == 0

    mesh = plsc.VectorSubcoreMesh(core_axis_name="c", subcore_axis_name="s")
    nc = info.num_cores

    @functools.partial(
        pl.kernel,
        out_type=jax.ShapeDtypeStruct((n_rows, _D), jnp.float32),
        mesh=mesh,
        compiler_params=pltpu.CompilerParams(use_tc_tiling_on_sc=False),
        scratch_types=[
            pltpu.VMEM((_RING, _GROUP), jnp.int32),
            pltpu.VMEM_SHARED((_N_COLS * _VOCAB, _D), jnp.float32),
            tuple(pltpu.VMEM((_GROUP, _D), jnp.float32) for _ in range(_RING)),
            tuple(pltpu.SemaphoreType.DMA for _ in range(_RING)),
            tuple(pltpu.SemaphoreType.DMA for _ in range(_RING)),
            tuple(pltpu.SemaphoreType.DMA for _ in range(_RING)),
            pltpu.SemaphoreType.DMA,
        ],
    )
    def gather_kernel(
        idx_hbm, table_hbm, out_hbm, idx_v, tab_sp, bufs, gsems, ssems,
        isems, tsem,
    ):
        wid = lax.axis_index("s") * nc + lax.axis_index("c")
        row_base = wid * rows_per_w

        sid = lax.axis_index("s")
        tab_rows = _N_COLS * _VOCAB // 16
        tab_sl = pl.ds(sid * tab_rows, tab_rows)
        pltpu.async_copy(table_hbm.at[tab_sl], tab_sp.at[tab_sl], tsem)

        lanes = lax.broadcasted_iota(jnp.int32, (_LANES,), 0)

        def start_idx(g, s):
            pltpu.async_copy(
                idx_hbm.at[pl.ds(row_base + g * _GROUP, _GROUP)],
                idx_v.at[s],
                isems[s],
            )

        def wait_idx(g, s):
            pltpu.make_async_copy(
                idx_hbm.at[pl.ds(row_base + g * _GROUP, _GROUP)],
                idx_v.at[s],
                isems[s],
            ).wait()

        def add_offsets(g, s):
            k0 = row_base + g * _GROUP
            for i in range(_GROUP // _LANES):
                pos = lanes + (k0 + i * _LANES)
                col = lax.rem(pos, _N_COLS)
                sl = pl.ds(i * _LANES, _LANES)
                idx_v[s, sl] = idx_v[s, sl] + col * _VOCAB

        def start_gather(g, s, b):
            pltpu.async_copy(tab_sp.at[idx_v.at[s]], bufs[b], gsems[b])

        def wait_gather(g, s, b):
            pltpu.make_async_copy(
                tab_sp.at[idx_v.at[s]], bufs[b], gsems[b]
            ).wait()

        def out_slice(g):
            return out_hbm.at[pl.ds(row_base + g * _GROUP, _GROUP)]

        def start_store(g, b):
            pltpu.async_copy(bufs[b], out_slice(g), ssems[b])

        def wait_store(g, b):
            pltpu.make_async_copy(bufs[b], out_slice(g), ssems[b]).wait()

        for g in range(_RING):
            start_idx(g, g)
        pltpu.make_async_copy(
            table_hbm.at[tab_sl], tab_sp.at[tab_sl], tsem
        ).wait()
        plsc.subcore_barrier()
        for g in range(_AHEAD):
            wait_idx(g, g)
            add_offsets(g, g)
            start_gather(g, g, g)

        def pipe_body(p, _):
            for b in range(_RING):
                g = p * _RING + b
                wait_gather(g, b, b)
                start_store(g, b)

                i2 = g + _RING

                @pl.when(i2 < n_groups)
                def _():
                    start_idx(i2, b)

                h = g + _AHEAD

                @pl.when(h < n_groups)
                def _():
                    hs = (b + _AHEAD) % _RING
                    hb = (b + _AHEAD) % _RING

                    @pl.when(h >= _RING)
                    def _():
                        wait_store(h - _RING, hb)

                    wait_idx(h, hs)
                    add_offsets(h, hs)
                    start_gather(h, hs, hb)

            return 0

        lax.fori_loop(0, n_groups // _RING, pipe_body, 0)

        for b in range(_RING):
            g = n_groups - _RING + b
            wait_store(g, b)

    return gather_kernel


def kernel(inputs, tables):
    b, l, c = inputs.shape
    n_rows = b * l * c
    idx_flat = inputs.astype(jnp.int32).reshape(n_rows)
    flat_tables = tables.reshape(c * tables.shape[1], tables.shape[2])
    out = _make_kernel(n_rows)(idx_flat, flat_tables)
    return out.reshape(b, l, c * _D)
